# baseline (speedup 1.0000x reference)
import functools

import jax
import jax.numpy as jnp
from jax import lax
from jax.experimental import pallas as pl
from jax.experimental.pallas import tpu as pltpu
from jax.experimental.pallas import tpu_sc as plsc

N = 10000
T = 2048
E = 320000
D = 128
O = 64

NC = 2
NS = 16
NW = NC * NS

C = 192
NB = 2
NCH0 = 68
NCH1 = 38
NCH_MAX = max(NCH0, NCH1)
E_PAD = NS * (NCH0 + NCH1) * C
DP = D // 2
CW = 16
T_PAD = ((T + 1 + NS * 8 - 1) // (NS * 8)) * (NS * 8)
RT = T_PAD // NS


def _sc_accumulate(xp, src_w, dst_w, zsum, zcnt, cw=CW):
    mesh = plsc.VectorSubcoreMesh(core_axis_name="c", subcore_axis_name="s",
                                  num_cores=NC, num_subcores=NS)

    @functools.partial(
        pl.kernel,
        out_type=(
            jax.ShapeDtypeStruct((NC, T_PAD, D), jnp.float32),
            jax.ShapeDtypeStruct((NC, T_PAD, cw), jnp.float32),
        ),
        mesh=mesh,
        compiler_params=pltpu.CompilerParams(use_tc_tiling_on_sc=False,
                                             needs_layout_passes=False),
        scratch_types=[
            pltpu.VMEM((NCH_MAX, C), jnp.int32),
            pltpu.VMEM((NCH_MAX, C), jnp.int32),
            [pltpu.VMEM((C, DP), jnp.int32)] * NB,
            [pltpu.VMEM((C, D), jnp.float32)] * NB,
            pltpu.VMEM((C, cw), jnp.float32),
            pltpu.VMEM_SHARED((T_PAD, D), jnp.float32),
            pltpu.VMEM_SHARED((T_PAD, cw), jnp.float32),
            pltpu.SemaphoreType.DMA,
            [pltpu.SemaphoreType.DMA] * NB,
            [pltpu.SemaphoreType.DMA] * NB,
        ],
    )
    def body(xp_hbm, src_hbm, dst_hbm, zsum_hbm, zcnt_hbm,
             sums_out, cnt_out, src_v, dst_v, gbuf, fbuf, ones_v,
             acc_sh, cnt_sh, sem_g, sems_s, sems_o):
        ci = lax.axis_index("c")
        si = lax.axis_index("s")
        ncho = jnp.where(ci == 0, NCH0, NCH1)
        off = jnp.where(ci == 0, si * NCH0, NS * NCH0 + si * NCH1)

        pltpu.sync_copy(zsum_hbm.at[pl.ds(si * RT, RT)],
                        acc_sh.at[pl.ds(si * RT, RT)])
        pltpu.sync_copy(zcnt_hbm.at[pl.ds(si * RT, RT)],
                        cnt_sh.at[pl.ds(si * RT, RT)])
        pltpu.sync_copy(src_hbm.at[pl.ds(off, NCH_MAX)], src_v)
        pltpu.sync_copy(dst_hbm.at[pl.ds(off, NCH_MAX)], dst_v)

        def fill(i, carry):
            ones_v[i // (cw // 16), pl.ds((i % (cw // 16)) * 16, 16)] = (
                jnp.ones((16,), jnp.float32))
            return carry

        lax.fori_loop(0, C * (cw // 16), fill, 0)
        plsc.subcore_barrier()

        def gather(j, b):
            pltpu.async_copy(xp_hbm.at[src_v.at[j]], gbuf[b], sem_g)

        def gather_wait(j, b):
            pltpu.make_async_copy(xp_hbm.at[src_v.at[j]], gbuf[b],
                                  sem_g).wait()

        def unpack(b):
            hi_mask = jnp.full((16,), -65536, jnp.int32)

            @plsc.parallel_loop(0, C, unroll=4)
            def _(r):
                for w in range(DP // 16):
                    v = gbuf[b][r, pl.ds(w * 16, 16)]
                    lo = plsc.bitcast(lax.shift_left(v, 16), jnp.float32)
                    hi = plsc.bitcast(lax.bitwise_and(v, hi_mask),
                                      jnp.float32)
                    fbuf[b][r, pl.ds(w * 16, 16)] = lo
                    fbuf[b][r, pl.ds(DP + w * 16, 16)] = hi

        def scatter(j, b):
            pltpu.async_copy(fbuf[b], acc_sh.at[dst_v.at[j]], sems_s[b],
                             add=True)
            pltpu.async_copy(ones_v, cnt_sh.at[dst_v.at[j]], sems_o[b],
                             add=True)

        def scatter_wait(j, b):
            pltpu.make_async_copy(fbuf[b], acc_sh.at[dst_v.at[j]],
                                  sems_s[b]).wait()
            pltpu.make_async_copy(ones_v, cnt_sh.at[dst_v.at[j]],
                                  sems_o[b]).wait()

        gather(0, 0)

        def group(g, carry):
            for b in range(NB):
                j = g * NB + b
                gather_wait(j, b)

                @pl.when(j + 1 < ncho)
                def _():
                    gather(j + 1, 1 - b)

                @pl.when(j >= 2)
                def _():
                    scatter_wait(j - 2, b)

                unpack(b)
                scatter(j, b)
            return carry

        lax.fori_loop(0, ncho // NB, group, 0)
        scatter_wait(ncho - 2, 0)
        scatter_wait(ncho - 1, 1)
        plsc.subcore_barrier()

        pltpu.sync_copy(acc_sh.at[pl.ds(si * RT, RT)],
                        sums_out.at[ci, pl.ds(si * RT, RT)])
        pltpu.sync_copy(cnt_sh.at[pl.ds(si * RT, RT)],
                        cnt_out.at[ci, pl.ds(si * RT, RT)])

    return body(xp, src_w, dst_w, zsum, zcnt)


def _tc_combine(sums_ref, cnt_ref, xt_ref, wl_ref, bl_ref, wr_ref, out_ref):
    s = sums_ref[0][:T] + sums_ref[1][:T]
    c = cnt_ref[0][:T, 0:1] + cnt_ref[1][:T, 0:1]
    mean = s / jnp.maximum(c, 1.0)
    h = lax.dot_general(mean, wl_ref[...],
                        (((1,), (1,)), ((), ())),
                        preferred_element_type=jnp.float32)
    h = h + bl_ref[...]
    h = h + lax.dot_general(xt_ref[...], wr_ref[...],
                            (((1,), (1,)), ((), ())),
                            preferred_element_type=jnp.float32)
    m = jnp.max(h, axis=-1, keepdims=True)
    e = h - m
    lse = jnp.log(jnp.sum(jnp.exp(e), axis=-1, keepdims=True))
    out_ref[...] = e - lse


def kernel(x, edge_index, num_target, W_l, b_l, W_r):
    del num_target
    src = edge_index[0]
    dst = edge_index[1]
    pad = E_PAD - E
    src_w = jnp.concatenate(
        [src, jnp.zeros((pad + NCH_MAX * C,), jnp.int32)]).reshape(-1, C)
    dst_w = jnp.concatenate(
        [dst, jnp.full((pad,), T, jnp.int32),
         jnp.zeros((NCH_MAX * C,), jnp.int32)]).reshape(-1, C)
    xb = x.astype(jnp.bfloat16)
    xp = lax.bitcast_convert_type(
        jnp.stack([xb[:, :DP], xb[:, DP:]], axis=-1), jnp.int32)
    zsum = jnp.zeros((T_PAD, D), jnp.float32)
    zcnt = jnp.zeros((T_PAD, CW), jnp.float32)
    sums, cnt = _sc_accumulate(xp, src_w, dst_w, zsum, zcnt)

    out = pl.pallas_call(
        _tc_combine,
        out_shape=jax.ShapeDtypeStruct((T, O), jnp.float32),
    )(sums, cnt, x[:T], W_l, b_l.reshape(1, O), W_r)
    return out


# speedup vs baseline: 1.0335x; 1.0335x over previous
import functools

import jax
import jax.numpy as jnp
from jax import lax
from jax.experimental import pallas as pl
from jax.experimental.pallas import tpu as pltpu
from jax.experimental.pallas import tpu_sc as plsc

N = 10000
T = 2048
E = 320000
D = 128
O = 64

NC = 2
NS = 16
NW = NC * NS

C = 192
NB = 2
NCH0 = 72
NCH1 = 34
NCH_MAX = max(NCH0, NCH1)
E_PAD = NS * (NCH0 + NCH1) * C
DP = D // 2
CW = 16
T_PAD = ((T + 1 + NS * 8 - 1) // (NS * 8)) * (NS * 8)
RT = T_PAD // NS


def _sc_accumulate(xp, src_w, dst_w, zsum, zcnt, cw=CW):
    mesh = plsc.VectorSubcoreMesh(core_axis_name="c", subcore_axis_name="s",
                                  num_cores=NC, num_subcores=NS)

    @functools.partial(
        pl.kernel,
        out_type=(
            jax.ShapeDtypeStruct((NC, T_PAD, D), jnp.float32),
            jax.ShapeDtypeStruct((NC, T_PAD, cw), jnp.float32),
        ),
        mesh=mesh,
        compiler_params=pltpu.CompilerParams(use_tc_tiling_on_sc=False,
                                             needs_layout_passes=False),
        scratch_types=[
            pltpu.VMEM((NCH_MAX, C), jnp.int32),
            pltpu.VMEM((NCH_MAX, C), jnp.int32),
            [pltpu.VMEM((C, DP), jnp.int32)] * NB,
            [pltpu.VMEM((C, D), jnp.float32)] * NB,
            pltpu.VMEM((C, cw), jnp.float32),
            pltpu.VMEM_SHARED((T_PAD, D), jnp.float32),
            pltpu.VMEM_SHARED((T_PAD, cw), jnp.float32),
            pltpu.SemaphoreType.DMA,
            [pltpu.SemaphoreType.DMA] * NB,
            [pltpu.SemaphoreType.DMA] * NB,
        ],
    )
    def body(xp_hbm, src_hbm, dst_hbm, zsum_hbm, zcnt_hbm,
             sums_out, cnt_out, src_v, dst_v, gbuf, fbuf, ones_v,
             acc_sh, cnt_sh, sem_g, sems_s, sems_o):
        ci = lax.axis_index("c")
        si = lax.axis_index("s")
        ncho = jnp.where(ci == 0, NCH0, NCH1)
        off = jnp.where(ci == 0, si * NCH0, NS * NCH0 + si * NCH1)

        pltpu.sync_copy(zsum_hbm.at[pl.ds(si * RT, RT)],
                        acc_sh.at[pl.ds(si * RT, RT)])
        pltpu.sync_copy(zcnt_hbm.at[pl.ds(si * RT, RT)],
                        cnt_sh.at[pl.ds(si * RT, RT)])
        pltpu.sync_copy(src_hbm.at[pl.ds(off, NCH_MAX)], src_v)
        pltpu.sync_copy(dst_hbm.at[pl.ds(off, NCH_MAX)], dst_v)

        def fill(i, carry):
            ones_v[i // (cw // 16), pl.ds((i % (cw // 16)) * 16, 16)] = (
                jnp.ones((16,), jnp.float32))
            return carry

        lax.fori_loop(0, C * (cw // 16), fill, 0)
        plsc.subcore_barrier()

        def gather(j, b):
            pltpu.async_copy(xp_hbm.at[src_v.at[j]], gbuf[b], sem_g)

        def gather_wait(j, b):
            pltpu.make_async_copy(xp_hbm.at[src_v.at[j]], gbuf[b],
                                  sem_g).wait()

        def unpack(b):
            hi_mask = jnp.full((16,), -65536, jnp.int32)

            @plsc.parallel_loop(0, C, unroll=4)
            def _(r):
                for w in range(DP // 16):
                    v = gbuf[b][r, pl.ds(w * 16, 16)]
                    lo = plsc.bitcast(lax.shift_left(v, 16), jnp.float32)
                    hi = plsc.bitcast(lax.bitwise_and(v, hi_mask),
                                      jnp.float32)
                    fbuf[b][r, pl.ds(w * 16, 16)] = lo
                    fbuf[b][r, pl.ds(DP + w * 16, 16)] = hi

        def scatter(j, b):
            pltpu.async_copy(fbuf[b], acc_sh.at[dst_v.at[j]], sems_s[b],
                             add=True)
            pltpu.async_copy(ones_v, cnt_sh.at[dst_v.at[j]], sems_o[b],
                             add=True)

        def scatter_wait(j, b):
            pltpu.make_async_copy(fbuf[b], acc_sh.at[dst_v.at[j]],
                                  sems_s[b]).wait()
            pltpu.make_async_copy(ones_v, cnt_sh.at[dst_v.at[j]],
                                  sems_o[b]).wait()

        gather(0, 0)

        def group(g, carry):
            for b in range(NB):
                j = g * NB + b
                gather_wait(j, b)

                @pl.when(j + 1 < ncho)
                def _():
                    gather(j + 1, 1 - b)

                @pl.when(j >= 2)
                def _():
                    scatter_wait(j - 2, b)

                unpack(b)
                scatter(j, b)
            return carry

        lax.fori_loop(0, ncho // NB, group, 0)
        scatter_wait(ncho - 2, 0)
        scatter_wait(ncho - 1, 1)
        plsc.subcore_barrier()

        pltpu.sync_copy(acc_sh.at[pl.ds(si * RT, RT)],
                        sums_out.at[ci, pl.ds(si * RT, RT)])
        pltpu.sync_copy(cnt_sh.at[pl.ds(si * RT, RT)],
                        cnt_out.at[ci, pl.ds(si * RT, RT)])

    return body(xp, src_w, dst_w, zsum, zcnt)


def _tc_combine(sums_ref, cnt_ref, xt_ref, wl_ref, bl_ref, wr_ref, out_ref):
    s = sums_ref[0][:T] + sums_ref[1][:T]
    c = cnt_ref[0][:T, 0:1] + cnt_ref[1][:T, 0:1]
    mean = s / jnp.maximum(c, 1.0)
    h = lax.dot_general(mean, wl_ref[...],
                        (((1,), (1,)), ((), ())),
                        preferred_element_type=jnp.float32)
    h = h + bl_ref[...]
    h = h + lax.dot_general(xt_ref[...], wr_ref[...],
                            (((1,), (1,)), ((), ())),
                            preferred_element_type=jnp.float32)
    m = jnp.max(h, axis=-1, keepdims=True)
    e = h - m
    lse = jnp.log(jnp.sum(jnp.exp(e), axis=-1, keepdims=True))
    out_ref[...] = e - lse


def kernel(x, edge_index, num_target, W_l, b_l, W_r):
    del num_target
    src = edge_index[0]
    dst = edge_index[1]
    pad = E_PAD - E
    src_w = jnp.concatenate(
        [src, jnp.zeros((pad + NCH_MAX * C,), jnp.int32)]).reshape(-1, C)
    dst_w = jnp.concatenate(
        [dst, jnp.full((pad,), T, jnp.int32),
         jnp.zeros((NCH_MAX * C,), jnp.int32)]).reshape(-1, C)
    xb = x.astype(jnp.bfloat16)
    xp = lax.bitcast_convert_type(
        jnp.stack([xb[:, :DP], xb[:, DP:]], axis=-1), jnp.int32)
    zsum = jnp.zeros((T_PAD, D), jnp.float32)
    zcnt = jnp.zeros((T_PAD, CW), jnp.float32)
    sums, cnt = _sc_accumulate(xp, src_w, dst_w, zsum, zcnt)

    out = pl.pallas_call(
        _tc_combine,
        out_shape=jax.ShapeDtypeStruct((T, O), jnp.float32),
    )(sums, cnt, x[:T], W_l, b_l.reshape(1, O), W_r)
    return out
